# Initial kernel scaffold; baseline (speedup 1.0000x reference)
import jax
import jax.numpy as jnp
from jax.experimental import pallas as pl


def kernel(x, edge_index, W1, b1, W2, b2):
    raise NotImplementedError("write your pallas kernel here")


# speedup vs baseline: 73.2739x; 73.2739x over previous
import jax
import jax.numpy as jnp
from jax import lax
from jax.experimental import pallas as pl
from jax.experimental.pallas import tpu as pltpu
from jax.experimental.pallas import tpu_sc as plsc

N = 100000
NPAD = 100352
SLICE = NPAD // 16
E = 6400000
KD = 8
NWD = 196
KP = 4
NWP = 392
EPAD = 32 * NWP * KP * 128
ROWS = EPAD // 128
RPW = ROWS // 32
NPADROWS = 352

_mesh = plsc.VectorSubcoreMesh(core_axis_name="c", subcore_axis_name="s")


def _deg_body(dst_hbm, zeros_hbm, out_hbm, idx_v, ones_v, acc_sh, ssem):
    c = lax.axis_index("c")
    s = lax.axis_index("s")
    w = c * 16 + s
    for i in range(8):
        ones_v[pl.ds(i * 16, 16)] = jnp.ones((16,), jnp.float32)
    pltpu.sync_copy(zeros_hbm.at[pl.ds(s * SLICE, SLICE)],
                    acc_sh.at[pl.ds(s * SLICE, SLICE)])
    plsc.subcore_barrier()

    def win(t, carry):
        row0 = w * RPW + t * KD
        pltpu.sync_copy(dst_hbm.at[pl.ds(row0, KD)], idx_v)
        ds_ = [pltpu.async_copy(ones_v, acc_sh.at[idx_v.at[j]], ssem, add=True)
               for j in range(KD)]
        for dsc in ds_:
            dsc.wait()
        return carry

    lax.fori_loop(0, NWD, win, 0)
    plsc.subcore_barrier()
    pltpu.sync_copy(acc_sh.at[pl.ds(s * SLICE, SLICE)],
                    out_hbm.at[c, pl.ds(s * SLICE, SLICE)])


_deg_call = pl.kernel(
    _deg_body,
    out_type=jax.ShapeDtypeStruct((2, NPAD), jnp.float32),
    mesh=_mesh,
    scratch_types=[
        pltpu.VMEM((KD, 128), jnp.int32),
        pltpu.VMEM((128,), jnp.float32),
        pltpu.VMEM_SHARED((NPAD,), jnp.float32),
        pltpu.SemaphoreType.DMA,
    ],
)


def _make_prop(width):
    def body(*refs):
        utabs = refs[:width]
        zeros_hbm, src_hbm, dst_hbm = refs[width:width + 3]
        outs = refs[width + 3:2 * width + 3]
        b = 2 * width + 3
        idxs_v, idxd_v = refs[b], refs[b + 1]
        vals = refs[b + 2:b + 2 + width]
        tabs = refs[b + 2 + width:b + 2 + 2 * width]
        accs = refs[b + 2 + 2 * width:b + 2 + 3 * width]
        gsem, ssem = refs[b + 2 + 3 * width], refs[b + 3 + 3 * width]
        c = lax.axis_index("c")
        s = lax.axis_index("s")
        w = c * 16 + s
        for k in range(width):
            pltpu.sync_copy(utabs[k].at[pl.ds(s * SLICE, SLICE)],
                            tabs[k].at[pl.ds(s * SLICE, SLICE)])
            pltpu.sync_copy(zeros_hbm.at[pl.ds(s * SLICE, SLICE)],
                            accs[k].at[pl.ds(s * SLICE, SLICE)])
        plsc.subcore_barrier()

        def win(t, carry):
            row0 = w * RPW + t * KP
            pltpu.sync_copy(src_hbm.at[pl.ds(row0, KP)], idxs_v)
            pltpu.sync_copy(dst_hbm.at[pl.ds(row0, KP)], idxd_v)
            for j in range(KP):
                gds = [pltpu.async_copy(tabs[k].at[idxs_v.at[j]], vals[k], gsem)
                       for k in range(width)]
                for dsc in gds:
                    dsc.wait()
                sds = [pltpu.async_copy(vals[k], accs[k].at[idxd_v.at[j]],
                                        ssem, add=True)
                       for k in range(width)]
                for dsc in sds:
                    dsc.wait()
            return carry

        lax.fori_loop(0, NWP, win, 0)
        plsc.subcore_barrier()
        for k in range(width):
            pltpu.sync_copy(accs[k].at[pl.ds(s * SLICE, SLICE)],
                            outs[k].at[c, pl.ds(s * SLICE, SLICE)])

    return pl.kernel(
        body,
        out_type=[jax.ShapeDtypeStruct((2, NPAD), jnp.float32)] * width,
        mesh=_mesh,
        scratch_types=(
            [pltpu.VMEM((KP, 128), jnp.int32)] * 2
            + [pltpu.VMEM((128,), jnp.float32)] * width
            + [pltpu.VMEM_SHARED((NPAD,), jnp.float32)] * (2 * width)
            + [pltpu.SemaphoreType.DMA] * 2
        ),
    )


_prop3_call = _make_prop(3)
_prop2_call = _make_prop(2)


RT, LT = NPAD // 128, 128


def _prep_body(degp, xp, d_out, u1_out):
    deg = 1.0 + degp[0] + degp[1]
    d = lax.rsqrt(deg)
    d_out[...] = d
    for k in range(3):
        u1_out[k] = xp[k] * d


def _sc(ref, i, j):
    return ref[i:i + 1, j:j + 1]


def _mid_body(a0, a1, a2, u1, d, w1t, b1, w2t, u2_out):
    dv = d[...]
    accs = (a0, a1, a2)
    p1 = [(u1[k] + accs[k][0] + accs[k][1]) * dv for k in range(3)]
    h = [jnp.maximum(sum(_sc(w1t, j, k) * p1[k] for k in range(3))
                     + _sc(b1, j, 0), 0.0)
         for j in range(16)]
    for i in range(2):
        g = sum(_sc(w2t, i, j) * h[j] for j in range(16))
        u2_out[i] = g * dv


def _final_body(a0, a1, u2, d, b2, out):
    dv = d[...]
    accs = (a0, a1)
    o = [(u2[i] + accs[i][0] + accs[i][1]) * dv + _sc(b2, i, 0)
         for i in range(2)]
    m = jnp.maximum(o[0], o[1])
    lse = m + jnp.log(jnp.exp(o[0] - m) + jnp.exp(o[1] - m))
    out[0] = o[0] - lse
    out[1] = o[1] - lse


_prep_call = pl.pallas_call(
    _prep_body,
    out_shape=[jax.ShapeDtypeStruct((RT, LT), jnp.float32),
               jax.ShapeDtypeStruct((3, RT, LT), jnp.float32)],
)

_mid_call = pl.pallas_call(
    _mid_body,
    out_shape=jax.ShapeDtypeStruct((2, RT, LT), jnp.float32),
)

_final_call = pl.pallas_call(
    _final_body,
    out_shape=jax.ShapeDtypeStruct((2, RT, LT), jnp.float32),
)


def kernel(x, edge_index, W1, b1, W2, b2):
    f32 = jnp.float32
    src = edge_index[0].astype(jnp.int32)
    dst = edge_index[1].astype(jnp.int32)
    pad_idx = (jnp.arange(EPAD - E, dtype=jnp.int32) % NPADROWS) + N
    src2d = jnp.concatenate([src, pad_idx]).reshape(ROWS, 128)
    dst2d = jnp.concatenate([dst, pad_idx]).reshape(ROWS, 128)

    zeros1 = jnp.zeros((NPAD,), f32)
    xp = jnp.zeros((3, NPAD), f32).at[:, :N].set(x.T)

    degp = _deg_call(dst2d, zeros1)
    d, u1 = _prep_call(degp.reshape(2, RT, LT), xp.reshape(3, RT, LT))
    u1f = u1.reshape(3, NPAD)
    acc1 = _prop3_call(u1f[0], u1f[1], u1f[2], zeros1, src2d, dst2d)
    u2 = _mid_call(*(a.reshape(2, RT, LT) for a in acc1), u1, d, W1.T,
                   b1.reshape(16, 1), W2.T)
    u2f = u2.reshape(2, NPAD)
    acc2 = _prop2_call(u2f[0], u2f[1], zeros1, src2d, dst2d)
    outf = _final_call(*(a.reshape(2, RT, LT) for a in acc2), u2, d,
                       b2.reshape(2, 1))
    return outf.reshape(2, NPAD)[:, :N].T


# speedup vs baseline: 84.8237x; 1.1576x over previous
import jax
import jax.numpy as jnp
from jax import lax
from jax.experimental import pallas as pl
from jax.experimental.pallas import tpu as pltpu
from jax.experimental.pallas import tpu_sc as plsc

N = 100000
NPAD = 100352
SLICE = NPAD // 16
E = 6400000
KD = 8
NWD = 196
KP = 4
NWP = 392
EPAD = 32 * NWP * KP * 128
ROWS = EPAD // 128
RPW = ROWS // 32
NPADROWS = 352

_mesh = plsc.VectorSubcoreMesh(core_axis_name="c", subcore_axis_name="s")


def _deg_body(dst_hbm, zeros_hbm, out_hbm, idx_v, ones_v, acc_sh, ssem):
    c = lax.axis_index("c")
    s = lax.axis_index("s")
    w = c * 16 + s
    for i in range(8):
        ones_v[pl.ds(i * 16, 16)] = jnp.ones((16,), jnp.float32)
    pltpu.sync_copy(zeros_hbm.at[pl.ds(s * SLICE, SLICE)],
                    acc_sh.at[pl.ds(s * SLICE, SLICE)])
    plsc.subcore_barrier()

    def win(t, carry):
        row0 = w * RPW + t * KD
        pltpu.sync_copy(dst_hbm.at[pl.ds(row0, KD)], idx_v)
        ds_ = [pltpu.async_copy(ones_v, acc_sh.at[idx_v.at[j]], ssem, add=True)
               for j in range(KD)]
        for dsc in ds_:
            dsc.wait()
        return carry

    lax.fori_loop(0, NWD, win, 0)
    plsc.subcore_barrier()
    pltpu.sync_copy(acc_sh.at[pl.ds(s * SLICE, SLICE)],
                    out_hbm.at[c, pl.ds(s * SLICE, SLICE)])


_deg_call = pl.kernel(
    _deg_body,
    out_type=jax.ShapeDtypeStruct((2, NPAD), jnp.float32),
    mesh=_mesh,
    scratch_types=[
        pltpu.VMEM((KD, 128), jnp.int32),
        pltpu.VMEM((128,), jnp.float32),
        pltpu.VMEM_SHARED((NPAD,), jnp.float32),
        pltpu.SemaphoreType.DMA,
    ],
)


def _make_prop(width):
    def body(*refs):
        utabs = refs[:width]
        zeros_hbm, src_hbm, dst_hbm = refs[width:width + 3]
        outs = refs[width + 3:2 * width + 3]
        b = 2 * width + 3
        nv = KP * width
        idxs_v, idxd_v = refs[b], refs[b + 1]
        vals = refs[b + 2:b + 2 + nv]
        tabs = refs[b + 2 + nv:b + 2 + nv + width]
        accs = refs[b + 2 + nv + width:b + 2 + nv + 2 * width]
        gsem, ssem = refs[b + 2 + nv + 2 * width], refs[b + 3 + nv + 2 * width]
        c = lax.axis_index("c")
        s = lax.axis_index("s")
        w = c * 16 + s
        for k in range(width):
            pltpu.sync_copy(utabs[k].at[pl.ds(s * SLICE, SLICE)],
                            tabs[k].at[pl.ds(s * SLICE, SLICE)])
            pltpu.sync_copy(zeros_hbm.at[pl.ds(s * SLICE, SLICE)],
                            accs[k].at[pl.ds(s * SLICE, SLICE)])
        plsc.subcore_barrier()

        def win(t, carry):
            row0 = w * RPW + t * KP
            pltpu.sync_copy(src_hbm.at[pl.ds(row0, KP)], idxs_v)
            pltpu.sync_copy(dst_hbm.at[pl.ds(row0, KP)], idxd_v)
            gds = [pltpu.async_copy(tabs[k].at[idxs_v.at[j]],
                                    vals[j * width + k], gsem)
                   for j in range(KP) for k in range(width)]
            for dsc in gds:
                dsc.wait()
            sds = [pltpu.async_copy(vals[j * width + k],
                                    accs[k].at[idxd_v.at[j]], ssem, add=True)
                   for j in range(KP) for k in range(width)]
            for dsc in sds:
                dsc.wait()
            return carry

        lax.fori_loop(0, NWP, win, 0)
        plsc.subcore_barrier()
        for k in range(width):
            pltpu.sync_copy(accs[k].at[pl.ds(s * SLICE, SLICE)],
                            outs[k].at[c, pl.ds(s * SLICE, SLICE)])

    return pl.kernel(
        body,
        out_type=[jax.ShapeDtypeStruct((2, NPAD), jnp.float32)] * width,
        mesh=_mesh,
        scratch_types=(
            [pltpu.VMEM((KP, 128), jnp.int32)] * 2
            + [pltpu.VMEM((128,), jnp.float32)] * (KP * width)
            + [pltpu.VMEM_SHARED((NPAD,), jnp.float32)] * (2 * width)
            + [pltpu.SemaphoreType.DMA] * 2
        ),
    )


_prop3_call = _make_prop(3)
_prop2_call = _make_prop(2)


RT, LT = NPAD // 128, 128


def _prep_body(degp, xp, d_out, u1_out):
    deg = 1.0 + degp[0] + degp[1]
    d = lax.rsqrt(deg)
    d_out[...] = d
    for k in range(3):
        u1_out[k] = xp[k] * d


def _sc(ref, i, j):
    return ref[i:i + 1, j:j + 1]


def _mid_body(a0, a1, a2, u1, d, w1t, b1, w2t, u2_out):
    dv = d[...]
    accs = (a0, a1, a2)
    p1 = [(u1[k] + accs[k][0] + accs[k][1]) * dv for k in range(3)]
    h = [jnp.maximum(sum(_sc(w1t, j, k) * p1[k] for k in range(3))
                     + _sc(b1, j, 0), 0.0)
         for j in range(16)]
    for i in range(2):
        g = sum(_sc(w2t, i, j) * h[j] for j in range(16))
        u2_out[i] = g * dv


def _final_body(a0, a1, u2, d, b2, out):
    dv = d[...]
    accs = (a0, a1)
    o = [(u2[i] + accs[i][0] + accs[i][1]) * dv + _sc(b2, i, 0)
         for i in range(2)]
    m = jnp.maximum(o[0], o[1])
    lse = m + jnp.log(jnp.exp(o[0] - m) + jnp.exp(o[1] - m))
    out[0] = o[0] - lse
    out[1] = o[1] - lse


_prep_call = pl.pallas_call(
    _prep_body,
    out_shape=[jax.ShapeDtypeStruct((RT, LT), jnp.float32),
               jax.ShapeDtypeStruct((3, RT, LT), jnp.float32)],
)

_mid_call = pl.pallas_call(
    _mid_body,
    out_shape=jax.ShapeDtypeStruct((2, RT, LT), jnp.float32),
)

_final_call = pl.pallas_call(
    _final_body,
    out_shape=jax.ShapeDtypeStruct((2, RT, LT), jnp.float32),
)


def kernel(x, edge_index, W1, b1, W2, b2):
    f32 = jnp.float32
    src = edge_index[0].astype(jnp.int32)
    dst = edge_index[1].astype(jnp.int32)
    pad_idx = (jnp.arange(EPAD - E, dtype=jnp.int32) % NPADROWS) + N
    src2d = jnp.concatenate([src, pad_idx]).reshape(ROWS, 128)
    dst2d = jnp.concatenate([dst, pad_idx]).reshape(ROWS, 128)

    zeros1 = jnp.zeros((NPAD,), f32)
    xp = jnp.zeros((3, NPAD), f32).at[:, :N].set(x.T)

    degp = _deg_call(dst2d, zeros1)
    d, u1 = _prep_call(degp.reshape(2, RT, LT), xp.reshape(3, RT, LT))
    u1f = u1.reshape(3, NPAD)
    acc1 = _prop3_call(u1f[0], u1f[1], u1f[2], zeros1, src2d, dst2d)
    u2 = _mid_call(*(a.reshape(2, RT, LT) for a in acc1), u1, d, W1.T,
                   b1.reshape(16, 1), W2.T)
    u2f = u2.reshape(2, NPAD)
    acc2 = _prop2_call(u2f[0], u2f[1], zeros1, src2d, dst2d)
    outf = _final_call(*(a.reshape(2, RT, LT) for a in acc2), u2, d,
                       b2.reshape(2, 1))
    return outf.reshape(2, NPAD)[:, :N].T


# speedup vs baseline: 130.7382x; 1.5413x over previous
import jax
import jax.numpy as jnp
from jax import lax
from jax.experimental import pallas as pl
from jax.experimental.pallas import tpu as pltpu
from jax.experimental.pallas import tpu_sc as plsc

N = 100000
NPAD = 100352
SLICE = NPAD // 16
E = 6400000
C = 2048
NCH = 98
NPAIR = NCH // 2
EPW = NCH * C
EPAD = 32 * EPW
NPADROWS = 352

_mesh = plsc.VectorSubcoreMesh(core_axis_name="c", subcore_axis_name="s")


def _wait_idx(dummy_hbm, buf, sem):
    pltpu.make_async_copy(dummy_hbm, buf, sem).wait()


def _deg_body(dst_hbm, ones_hbm, zeros_hbm, out_hbm, db0, db1, ones_v,
              acc_sh, isem, ssem):
    c = lax.axis_index("c")
    s = lax.axis_index("s")
    w = c * 16 + s
    base = w * EPW
    pltpu.sync_copy(ones_hbm, ones_v)
    pltpu.sync_copy(zeros_hbm.at[pl.ds(s * SLICE, SLICE)],
                    acc_sh.at[pl.ds(s * SLICE, SLICE)])
    plsc.subcore_barrier()
    pltpu.async_copy(dst_hbm.at[pl.ds(base, C)], db0, isem)

    def pair(i, carry):
        g0 = base + (2 * i) * C
        _wait_idx(dst_hbm.at[pl.ds(0, C)], db0, isem)
        pltpu.async_copy(dst_hbm.at[pl.ds(g0 + C, C)], db1, isem)
        s0 = pltpu.async_copy(ones_v, acc_sh.at[db0], ssem, add=True)
        _wait_idx(dst_hbm.at[pl.ds(0, C)], db1, isem)
        s0.wait()
        pltpu.async_copy(dst_hbm.at[pl.ds(g0 + 2 * C, C)], db0, isem)
        pltpu.async_copy(ones_v, acc_sh.at[db1], ssem, add=True).wait()
        return carry

    lax.fori_loop(0, NPAIR, pair, 0)
    _wait_idx(dst_hbm.at[pl.ds(0, C)], db0, isem)
    plsc.subcore_barrier()
    pltpu.sync_copy(acc_sh.at[pl.ds(s * SLICE, SLICE)],
                    out_hbm.at[c, pl.ds(s * SLICE, SLICE)])


_deg_call = pl.kernel(
    _deg_body,
    out_type=jax.ShapeDtypeStruct((2, NPAD), jnp.float32),
    mesh=_mesh,
    scratch_types=[
        pltpu.VMEM((C,), jnp.int32),
        pltpu.VMEM((C,), jnp.int32),
        pltpu.VMEM((C,), jnp.float32),
        pltpu.VMEM_SHARED((NPAD,), jnp.float32),
        pltpu.SemaphoreType.DMA,
        pltpu.SemaphoreType.DMA,
    ],
)


def _make_prop(width):
    def body(*refs):
        utabs = refs[:width]
        zeros_hbm, src_hbm, dst_hbm = refs[width:width + 3]
        outs = refs[width + 3:2 * width + 3]
        b = 2 * width + 3
        sb0, db0, sb1, db1 = refs[b:b + 4]
        vals0 = refs[b + 4:b + 4 + width]
        vals1 = refs[b + 4 + width:b + 4 + 2 * width]
        tabs = refs[b + 4 + 2 * width:b + 4 + 3 * width]
        accs = refs[b + 4 + 3 * width:b + 4 + 4 * width]
        isem, gsem, ssem = refs[b + 4 + 4 * width:b + 7 + 4 * width]
        c = lax.axis_index("c")
        s = lax.axis_index("s")
        w = c * 16 + s
        base = w * EPW
        for k in range(width):
            pltpu.sync_copy(utabs[k].at[pl.ds(s * SLICE, SLICE)],
                            tabs[k].at[pl.ds(s * SLICE, SLICE)])
            pltpu.sync_copy(zeros_hbm.at[pl.ds(s * SLICE, SLICE)],
                            accs[k].at[pl.ds(s * SLICE, SLICE)])
        plsc.subcore_barrier()
        pltpu.async_copy(src_hbm.at[pl.ds(base, C)], sb0, isem)
        pltpu.async_copy(dst_hbm.at[pl.ds(base, C)], db0, isem)

        def pair(i, carry):
            g0 = base + (2 * i) * C
            _wait_idx(src_hbm.at[pl.ds(0, C)], sb0, isem)
            _wait_idx(src_hbm.at[pl.ds(0, C)], db0, isem)
            pltpu.async_copy(src_hbm.at[pl.ds(g0 + C, C)], sb1, isem)
            pltpu.async_copy(dst_hbm.at[pl.ds(g0 + C, C)], db1, isem)
            g0d = [pltpu.async_copy(tabs[k].at[sb0], vals0[k], gsem)
                   for k in range(width)]
            for dsc in g0d:
                dsc.wait()
            s0d = [pltpu.async_copy(vals0[k], accs[k].at[db0], ssem, add=True)
                   for k in range(width)]
            _wait_idx(src_hbm.at[pl.ds(0, C)], sb1, isem)
            _wait_idx(src_hbm.at[pl.ds(0, C)], db1, isem)
            g1d = [pltpu.async_copy(tabs[k].at[sb1], vals1[k], gsem)
                   for k in range(width)]
            for dsc in s0d:
                dsc.wait()
            pltpu.async_copy(src_hbm.at[pl.ds(g0 + 2 * C, C)], sb0, isem)
            pltpu.async_copy(dst_hbm.at[pl.ds(g0 + 2 * C, C)], db0, isem)
            for dsc in g1d:
                dsc.wait()
            s1d = [pltpu.async_copy(vals1[k], accs[k].at[db1], ssem, add=True)
                   for k in range(width)]
            for dsc in s1d:
                dsc.wait()
            return carry

        lax.fori_loop(0, NPAIR, pair, 0)
        _wait_idx(src_hbm.at[pl.ds(0, C)], sb0, isem)
        _wait_idx(src_hbm.at[pl.ds(0, C)], db0, isem)
        plsc.subcore_barrier()
        for k in range(width):
            pltpu.sync_copy(accs[k].at[pl.ds(s * SLICE, SLICE)],
                            outs[k].at[c, pl.ds(s * SLICE, SLICE)])

    return pl.kernel(
        body,
        out_type=[jax.ShapeDtypeStruct((2, NPAD), jnp.float32)] * width,
        mesh=_mesh,
        scratch_types=(
            [pltpu.VMEM((C,), jnp.int32)] * 4
            + [pltpu.VMEM((C,), jnp.float32)] * (2 * width)
            + [pltpu.VMEM_SHARED((NPAD,), jnp.float32)] * (2 * width)
            + [pltpu.SemaphoreType.DMA] * 3
        ),
    )


_prop3_call = _make_prop(3)
_prop2_call = _make_prop(2)


RT, LT = NPAD // 128, 128


def _prep_body(degp, xp, d_out, u1_out):
    deg = 1.0 + degp[0] + degp[1]
    d = lax.rsqrt(deg)
    d_out[...] = d
    for k in range(3):
        u1_out[k] = xp[k] * d


def _sc(ref, i, j):
    return ref[i:i + 1, j:j + 1]


def _mid_body(a0, a1, a2, u1, d, w1t, b1, w2t, u2_out):
    dv = d[...]
    accs = (a0, a1, a2)
    p1 = [(u1[k] + accs[k][0] + accs[k][1]) * dv for k in range(3)]
    h = [jnp.maximum(sum(_sc(w1t, j, k) * p1[k] for k in range(3))
                     + _sc(b1, j, 0), 0.0)
         for j in range(16)]
    for i in range(2):
        g = sum(_sc(w2t, i, j) * h[j] for j in range(16))
        u2_out[i] = g * dv


def _final_body(a0, a1, u2, d, b2, out):
    dv = d[...]
    accs = (a0, a1)
    o = [(u2[i] + accs[i][0] + accs[i][1]) * dv + _sc(b2, i, 0)
         for i in range(2)]
    m = jnp.maximum(o[0], o[1])
    lse = m + jnp.log(jnp.exp(o[0] - m) + jnp.exp(o[1] - m))
    out[0] = o[0] - lse
    out[1] = o[1] - lse


_prep_call = pl.pallas_call(
    _prep_body,
    out_shape=[jax.ShapeDtypeStruct((RT, LT), jnp.float32),
               jax.ShapeDtypeStruct((3, RT, LT), jnp.float32)],
)

_mid_call = pl.pallas_call(
    _mid_body,
    out_shape=jax.ShapeDtypeStruct((2, RT, LT), jnp.float32),
)

_final_call = pl.pallas_call(
    _final_body,
    out_shape=jax.ShapeDtypeStruct((2, RT, LT), jnp.float32),
)


def kernel(x, edge_index, W1, b1, W2, b2):
    f32 = jnp.float32
    src = edge_index[0].astype(jnp.int32)
    dst = edge_index[1].astype(jnp.int32)
    pad_idx = (jnp.arange(EPAD - E + C, dtype=jnp.int32) % NPADROWS) + N
    src1 = jnp.concatenate([src, pad_idx])
    dst1 = jnp.concatenate([dst, pad_idx])

    zeros1 = jnp.zeros((NPAD,), f32)
    ones_c = jnp.ones((C,), f32)
    xp = jnp.zeros((3, NPAD), f32).at[:, :N].set(x.T)

    degp = _deg_call(dst1, ones_c, zeros1)
    d, u1 = _prep_call(degp.reshape(2, RT, LT), xp.reshape(3, RT, LT))
    u1f = u1.reshape(3, NPAD)
    acc1 = _prop3_call(u1f[0], u1f[1], u1f[2], zeros1, src1, dst1)
    u2 = _mid_call(*(a.reshape(2, RT, LT) for a in acc1), u1, d, W1.T,
                   b1.reshape(16, 1), W2.T)
    u2f = u2.reshape(2, NPAD)
    acc2 = _prop2_call(u2f[0], u2f[1], zeros1, src1, dst1)
    outf = _final_call(*(a.reshape(2, RT, LT) for a in acc2), u2, d,
                       b2.reshape(2, 1))
    return outf.reshape(2, NPAD)[:, :N].T


# speedup vs baseline: 186.0820x; 1.4233x over previous
import jax
import jax.numpy as jnp
from jax import lax
from jax.experimental import pallas as pl
from jax.experimental.pallas import tpu as pltpu
from jax.experimental.pallas import tpu_sc as plsc

N = 100000
NPAD = 100352
SLICE = NPAD // 16
E = 6400000
C = 2048
NCH = 98
NPAIR = NCH // 2
EPW = NCH * C
EPAD = 32 * EPW
NPADROWS = 352

_mesh = plsc.VectorSubcoreMesh(core_axis_name="c", subcore_axis_name="s")


def _wait_idx(dummy_hbm, buf, sem):
    pltpu.make_async_copy(dummy_hbm, buf, sem).wait()


def _deg_body(dst_hbm, ones_hbm, zeros_hbm, out_hbm, db0, db1, ones_v,
              acc_sh, isem, ssem):
    c = lax.axis_index("c")
    s = lax.axis_index("s")
    w = c * 16 + s
    base = w * EPW
    pltpu.sync_copy(ones_hbm, ones_v)
    pltpu.sync_copy(zeros_hbm.at[pl.ds(s * SLICE, SLICE)],
                    acc_sh.at[pl.ds(s * SLICE, SLICE)])
    plsc.subcore_barrier()
    pltpu.async_copy(dst_hbm.at[pl.ds(base, C)], db0, isem)

    def pair(i, carry):
        g0 = base + (2 * i) * C
        _wait_idx(dst_hbm.at[pl.ds(0, C)], db0, isem)
        pltpu.async_copy(dst_hbm.at[pl.ds(g0 + C, C)], db1, isem)
        s0 = pltpu.async_copy(ones_v, acc_sh.at[db0], ssem, add=True)
        _wait_idx(dst_hbm.at[pl.ds(0, C)], db1, isem)
        s0.wait()
        pltpu.async_copy(dst_hbm.at[pl.ds(g0 + 2 * C, C)], db0, isem)
        pltpu.async_copy(ones_v, acc_sh.at[db1], ssem, add=True).wait()
        return carry

    lax.fori_loop(0, NPAIR, pair, 0)
    _wait_idx(dst_hbm.at[pl.ds(0, C)], db0, isem)
    plsc.subcore_barrier()
    pltpu.sync_copy(acc_sh.at[pl.ds(s * SLICE, SLICE)],
                    out_hbm.at[c, pl.ds(s * SLICE, SLICE)])


_deg_call = pl.kernel(
    _deg_body,
    out_type=jax.ShapeDtypeStruct((2, NPAD), jnp.float32),
    mesh=_mesh,
    scratch_types=[
        pltpu.VMEM((C,), jnp.int32),
        pltpu.VMEM((C,), jnp.int32),
        pltpu.VMEM((C,), jnp.float32),
        pltpu.VMEM_SHARED((NPAD,), jnp.float32),
        pltpu.SemaphoreType.DMA,
        pltpu.SemaphoreType.DMA,
    ],
)


def _make_prop(width):
    def body(*refs):
        utabs = refs[:width]
        zeros_hbm, src_hbm, dst_hbm = refs[width:width + 3]
        outs = refs[width + 3:2 * width + 3]
        b = 2 * width + 3
        sb0, db0, sb1, db1 = refs[b:b + 4]
        vals0 = refs[b + 4:b + 4 + width]
        vals1 = refs[b + 4 + width:b + 4 + 2 * width]
        tabs = refs[b + 4 + 2 * width:b + 4 + 3 * width]
        accs = refs[b + 4 + 3 * width:b + 4 + 4 * width]
        isem, gsem, ssem = refs[b + 4 + 4 * width:b + 7 + 4 * width]
        c = lax.axis_index("c")
        s = lax.axis_index("s")
        w = c * 16 + s
        base = w * EPW
        for k in range(width):
            pltpu.sync_copy(utabs[k].at[pl.ds(s * SLICE, SLICE)],
                            tabs[k].at[pl.ds(s * SLICE, SLICE)])
            pltpu.sync_copy(zeros_hbm.at[pl.ds(s * SLICE, SLICE)],
                            accs[k].at[pl.ds(s * SLICE, SLICE)])
        plsc.subcore_barrier()
        pltpu.sync_copy(src_hbm.at[pl.ds(base, C)], sb0)
        pltpu.sync_copy(dst_hbm.at[pl.ds(base, C)], db0)
        for k in range(width):
            pltpu.async_copy(tabs[k].at[sb0], vals0[k], gsem)
        pltpu.async_copy(src_hbm.at[pl.ds(base + C, C)], sb1, isem)
        pltpu.async_copy(dst_hbm.at[pl.ds(base + C, C)], db1, isem)

        def pair(i, carry):
            g0 = base + (2 * i) * C
            for k in range(width):
                pltpu.make_async_copy(zeros_hbm.at[pl.ds(0, C)],
                                      vals0[k], gsem).wait()
            s0d = [pltpu.async_copy(vals0[k], accs[k].at[db0], ssem, add=True)
                   for k in range(width)]
            _wait_idx(src_hbm.at[pl.ds(0, C)], sb1, isem)
            _wait_idx(src_hbm.at[pl.ds(0, C)], db1, isem)
            g1d = [pltpu.async_copy(tabs[k].at[sb1], vals1[k], gsem)
                   for k in range(width)]
            for dsc in s0d:
                dsc.wait()
            pltpu.async_copy(src_hbm.at[pl.ds(g0 + 2 * C, C)], sb0, isem)
            pltpu.async_copy(dst_hbm.at[pl.ds(g0 + 2 * C, C)], db0, isem)
            for dsc in g1d:
                dsc.wait()
            s1d = [pltpu.async_copy(vals1[k], accs[k].at[db1], ssem, add=True)
                   for k in range(width)]
            _wait_idx(src_hbm.at[pl.ds(0, C)], sb0, isem)
            _wait_idx(src_hbm.at[pl.ds(0, C)], db0, isem)
            for k in range(width):
                pltpu.async_copy(tabs[k].at[sb0], vals0[k], gsem)
            for dsc in s1d:
                dsc.wait()
            pltpu.async_copy(src_hbm.at[pl.ds(g0 + 3 * C, C)], sb1, isem)
            pltpu.async_copy(dst_hbm.at[pl.ds(g0 + 3 * C, C)], db1, isem)
            return carry

        lax.fori_loop(0, NPAIR, pair, 0)
        for k in range(width):
            pltpu.make_async_copy(zeros_hbm.at[pl.ds(0, C)],
                                  vals0[k], gsem).wait()
        _wait_idx(src_hbm.at[pl.ds(0, C)], sb1, isem)
        _wait_idx(src_hbm.at[pl.ds(0, C)], db1, isem)
        plsc.subcore_barrier()
        for k in range(width):
            pltpu.sync_copy(accs[k].at[pl.ds(s * SLICE, SLICE)],
                            outs[k].at[c, pl.ds(s * SLICE, SLICE)])

    return pl.kernel(
        body,
        out_type=[jax.ShapeDtypeStruct((2, NPAD), jnp.float32)] * width,
        mesh=_mesh,
        scratch_types=(
            [pltpu.VMEM((C,), jnp.int32)] * 4
            + [pltpu.VMEM((C,), jnp.float32)] * (2 * width)
            + [pltpu.VMEM_SHARED((NPAD,), jnp.float32)] * (2 * width)
            + [pltpu.SemaphoreType.DMA] * 3
        ),
    )


_prop3_call = _make_prop(3)
_prop2_call = _make_prop(2)


RT, LT = NPAD // 128, 128


def _prep_body(degp, xp, d_out, u1_out):
    deg = 1.0 + degp[0] + degp[1]
    d = lax.rsqrt(deg)
    d_out[...] = d
    for k in range(3):
        u1_out[k] = xp[k] * d


def _sc(ref, i, j):
    return ref[i:i + 1, j:j + 1]


def _mid_body(a0, a1, a2, u1, d, w1t, b1, w2t, u2_out):
    dv = d[...]
    accs = (a0, a1, a2)
    p1 = [(u1[k] + accs[k][0] + accs[k][1]) * dv for k in range(3)]
    h = [jnp.maximum(sum(_sc(w1t, j, k) * p1[k] for k in range(3))
                     + _sc(b1, j, 0), 0.0)
         for j in range(16)]
    for i in range(2):
        g = sum(_sc(w2t, i, j) * h[j] for j in range(16))
        u2_out[i] = g * dv


def _final_body(a0, a1, u2, d, b2, out):
    dv = d[...]
    accs = (a0, a1)
    o = [(u2[i] + accs[i][0] + accs[i][1]) * dv + _sc(b2, i, 0)
         for i in range(2)]
    m = jnp.maximum(o[0], o[1])
    lse = m + jnp.log(jnp.exp(o[0] - m) + jnp.exp(o[1] - m))
    out[0] = o[0] - lse
    out[1] = o[1] - lse


_prep_call = pl.pallas_call(
    _prep_body,
    out_shape=[jax.ShapeDtypeStruct((RT, LT), jnp.float32),
               jax.ShapeDtypeStruct((3, RT, LT), jnp.float32)],
)

_mid_call = pl.pallas_call(
    _mid_body,
    out_shape=jax.ShapeDtypeStruct((2, RT, LT), jnp.float32),
)

_final_call = pl.pallas_call(
    _final_body,
    out_shape=jax.ShapeDtypeStruct((2, RT, LT), jnp.float32),
)


def kernel(x, edge_index, W1, b1, W2, b2):
    f32 = jnp.float32
    src = edge_index[0].astype(jnp.int32)
    dst = edge_index[1].astype(jnp.int32)
    pad_idx = (jnp.arange(EPAD - E + 2 * C, dtype=jnp.int32) % NPADROWS) + N
    src1 = jnp.concatenate([src, pad_idx])
    dst1 = jnp.concatenate([dst, pad_idx])

    zeros1 = jnp.zeros((NPAD,), f32)
    ones_c = jnp.ones((C,), f32)
    xp = jnp.zeros((3, NPAD), f32).at[:, :N].set(x.T)

    degp = _deg_call(dst1, ones_c, zeros1)
    d, u1 = _prep_call(degp.reshape(2, RT, LT), xp.reshape(3, RT, LT))
    u1f = u1.reshape(3, NPAD)
    acc1 = _prop3_call(u1f[0], u1f[1], u1f[2], zeros1, src1, dst1)
    u2 = _mid_call(*(a.reshape(2, RT, LT) for a in acc1), u1, d, W1.T,
                   b1.reshape(16, 1), W2.T)
    u2f = u2.reshape(2, NPAD)
    acc2 = _prop2_call(u2f[0], u2f[1], zeros1, src1, dst1)
    outf = _final_call(*(a.reshape(2, RT, LT) for a in acc2), u2, d,
                       b2.reshape(2, 1))
    return outf.reshape(2, NPAD)[:, :N].T
